# Initial kernel scaffold; baseline (speedup 1.0000x reference)
import jax
import jax.numpy as jnp
from jax.experimental import pallas as pl


def kernel(x, real, imag, edge_index, edge_weight, params):
    raise NotImplementedError("write your pallas kernel here")


# speedup vs baseline: 3.1763x; 3.1763x over previous
import functools

import jax
import jax.numpy as jnp
from jax import lax
from jax.experimental import pallas as pl
from jax.experimental.pallas import tpu as pltpu
from jax.experimental.pallas import tpu_sc as plsc

N = 10000
E = 160000
N_PAD = 10240
NC, NS, NW = 2, 16, 32
B = 128
Q = 0.25

E_PAD = 163840
EH_PAD = 327680

_mesh = plsc.VectorSubcoreMesh(
    core_axis_name="c", subcore_axis_name="s", num_cores=NC, num_subcores=NS)


def _wid():
    return lax.axis_index("s") * NC + lax.axis_index("c")


def _zero_fill(buf, rows, width):
    z = jnp.zeros((16,), jnp.float32)

    def body(r, _):
        for f in range(width // 16):
            buf[r, pl.ds(f * 16, 16)] = z
        return 0
    lax.fori_loop(0, rows, body, 0)


def _zero_fill_1d(buf, n):
    z = jnp.zeros((16,), jnp.float32)

    def body(i, _):
        buf[pl.ds(i * 16, 16)] = z
        return 0
    lax.fori_loop(0, n // 16, body, 0)


_NBLK_E = E_PAD // (NW * B)


@functools.partial(
    pl.kernel,
    out_type=jax.ShapeDtypeStruct((NC, 2, N_PAD), jnp.float32),
    mesh=_mesh,
    compiler_params=pltpu.CompilerParams(needs_layout_passes=False),
    scratch_types=[
        pltpu.VMEM((B,), jnp.int32),
        pltpu.VMEM((B,), jnp.int32),
        pltpu.VMEM((B,), jnp.float32),
        pltpu.VMEM((B,), jnp.float32),
        pltpu.VMEM((B,), jnp.float32),
        pltpu.VMEM_SHARED((N_PAD,), jnp.float32),
        pltpu.VMEM_SHARED((N_PAD,), jnp.float32),
    ],
)
def _sc_degrees(src_h, dst_h, wsym_h, ew_h, out_h,
                sidx, didx, wm, wg, zbuf, accm, accg):
    c = lax.axis_index("c")
    s = lax.axis_index("s")
    w = _wid()
    rpt = N_PAD // NS
    _zero_fill_1d(zbuf, B)

    def zb(i, _):
        pltpu.sync_copy(zbuf, accm.at[pl.ds(s * rpt + i * B, B)])
        pltpu.sync_copy(zbuf, accg.at[pl.ds(s * rpt + i * B, B)])
        return 0
    lax.fori_loop(0, rpt // B, zb, 0)
    plsc.subcore_barrier()

    def body(b, _):
        base = (w * _NBLK_E + b) * B
        pltpu.sync_copy(src_h.at[pl.ds(base, B)], sidx)
        pltpu.sync_copy(dst_h.at[pl.ds(base, B)], didx)
        pltpu.sync_copy(wsym_h.at[pl.ds(base, B)], wm)
        pltpu.sync_copy(ew_h.at[pl.ds(base, B)], wg)
        pltpu.sync_copy(wm, accm.at[sidx], add=True)
        pltpu.sync_copy(wm, accm.at[didx], add=True)
        pltpu.sync_copy(wg, accg.at[didx], add=True)
        return 0
    lax.fori_loop(0, _NBLK_E, body, 0)
    plsc.subcore_barrier()

    def cp(i, _):
        off = s * rpt + i * B
        pltpu.sync_copy(accm.at[pl.ds(off, B)], out_h.at[c, 0, pl.ds(off, B)])
        pltpu.sync_copy(accg.at[pl.ds(off, B)], out_h.at[c, 1, pl.ds(off, B)])
        return 0
    lax.fori_loop(0, rpt // B, cp, 0)


@functools.partial(
    pl.kernel,
    out_type=[jax.ShapeDtypeStruct((E_PAD,), jnp.float32) for _ in range(4)],
    mesh=_mesh,
    compiler_params=pltpu.CompilerParams(needs_layout_passes=False),
    scratch_types=[
        pltpu.VMEM((N_PAD // 128, 128), jnp.float32),
        pltpu.VMEM((N_PAD // 128, 128), jnp.float32),
        pltpu.VMEM((B,), jnp.int32),
        pltpu.VMEM((B,), jnp.int32),
        pltpu.VMEM((B,), jnp.float32),
        pltpu.VMEM((B,), jnp.float32),
        pltpu.VMEM((B,), jnp.float32),
        pltpu.VMEM((B,), jnp.float32),
        pltpu.VMEM((B,), jnp.float32),
        pltpu.VMEM((B,), jnp.float32),
        pltpu.VMEM((B,), jnp.float32),
        pltpu.VMEM((B,), jnp.float32),
    ],
)
def _sc_norms(src_h, dst_h, wsym_h, ew_h, cos_h, sin_h, dm_h, dg_h,
              nr_o, ni_o, nin_o, gn_o,
              dmt, dgt, sidx, didx, wm, wg, cth, sth, bnr, bni, bnin, bgn):
    w = _wid()
    pltpu.sync_copy(dm_h, dmt)
    pltpu.sync_copy(dg_h, dgt)

    def body(b, _):
        base = (w * _NBLK_E + b) * B
        pltpu.sync_copy(src_h.at[pl.ds(base, B)], sidx)
        pltpu.sync_copy(dst_h.at[pl.ds(base, B)], didx)
        pltpu.sync_copy(wsym_h.at[pl.ds(base, B)], wm)
        pltpu.sync_copy(ew_h.at[pl.ds(base, B)], wg)
        pltpu.sync_copy(cos_h.at[pl.ds(base, B)], cth)
        pltpu.sync_copy(sin_h.at[pl.ds(base, B)], sth)

        def grp(g, _):
            o = g * 16
            s16 = sidx[pl.ds(o, 16)]
            d16 = didx[pl.ds(o, 16)]
            srow = lax.shift_right_logical(s16, 7)
            scol = lax.bitwise_and(s16, 127)
            drow = lax.shift_right_logical(d16, 7)
            dcol = lax.bitwise_and(d16, 127)
            dms = plsc.load_gather(dmt, [srow, scol])
            dmd = plsc.load_gather(dmt, [drow, dcol])
            dgs = plsc.load_gather(dgt, [srow, scol])
            dgd = plsc.load_gather(dgt, [drow, dcol])
            nrm = dms * wm[pl.ds(o, 16)] * dmd
            nr = -nrm * cth[pl.ds(o, 16)]
            ni = -nrm * sth[pl.ds(o, 16)]
            bnr[pl.ds(o, 16)] = nr
            bni[pl.ds(o, 16)] = ni
            bnin[pl.ds(o, 16)] = -ni
            bgn[pl.ds(o, 16)] = dgd * wg[pl.ds(o, 16)] * dgs
            return 0
        lax.fori_loop(0, B // 16, grp, 0)
        pltpu.sync_copy(bnr, nr_o.at[pl.ds(base, B)])
        pltpu.sync_copy(bni, ni_o.at[pl.ds(base, B)])
        pltpu.sync_copy(bnin, nin_o.at[pl.ds(base, B)])
        pltpu.sync_copy(bgn, gn_o.at[pl.ds(base, B)])
        return 0
    lax.fori_loop(0, _NBLK_E, body, 0)


def _make_prop(real_w, n_edges, is_complex):
    nblk = n_edges // (NW * B)
    dc = real_w // 2
    W128 = 128

    @functools.partial(
        pl.kernel,
        out_type=jax.ShapeDtypeStruct((NC, N_PAD, W128), jnp.float32),
        mesh=_mesh,
        compiler_params=pltpu.CompilerParams(needs_layout_passes=False),
        scratch_types=[
            pltpu.VMEM((B,), jnp.int32),
            pltpu.VMEM((B,), jnp.int32),
            pltpu.VMEM((B,), jnp.float32),
            pltpu.VMEM((B,), jnp.float32),
            pltpu.VMEM((B, W128), jnp.float32),
            pltpu.VMEM((B, W128), jnp.float32),
            pltpu.VMEM_SHARED((N_PAD, W128), jnp.float32),
            pltpu.SemaphoreType.DMA,
        ],
    )
    def k(xf_h, rows_h, cols_h, wr_h, wi_h, out_h,
          cidx, ridx, wrv, wiv, xbuf, obuf, acc, sem):
        c = lax.axis_index("c")
        s = lax.axis_index("s")
        w = _wid()
        rpt = N_PAD // NS
        _zero_fill(obuf, B, W128)

        def zb(i, _):
            pltpu.sync_copy(obuf, acc.at[pl.ds(s * rpt + i * B, B)])
            return 0
        lax.fori_loop(0, rpt // B, zb, 0)
        plsc.subcore_barrier()

        def body(b, _):
            base = (w * nblk + b) * B
            pltpu.sync_copy(cols_h.at[pl.ds(base, B)], cidx)
            pltpu.sync_copy(rows_h.at[pl.ds(base, B)], ridx)
            pltpu.sync_copy(wr_h.at[pl.ds(base, B)], wrv)
            if is_complex:
                pltpu.sync_copy(wi_h.at[pl.ds(base, B)], wiv)
            pltpu.async_copy(xf_h.at[cidx], xbuf, sem).wait()

            def grp(g, _):
                e16 = g * 16 + lax.iota(jnp.int32, 16)
                wr16 = wrv[pl.ds(g * 16, 16)]
                if is_complex:
                    wi16 = wiv[pl.ds(g * 16, 16)]
                    for f in range(dc):
                        fv = jnp.full((16,), f, jnp.int32)
                        fv2 = jnp.full((16,), f + dc, jnp.int32)
                        xr = plsc.load_gather(xbuf, [e16, fv])
                        xi = plsc.load_gather(xbuf, [e16, fv2])
                        plsc.store_scatter(obuf, [e16, fv],
                                           wr16 * xr - wi16 * xi)
                        plsc.store_scatter(obuf, [e16, fv2],
                                           wi16 * xr + wr16 * xi)
                else:
                    for f in range(real_w):
                        fv = jnp.full((16,), f, jnp.int32)
                        xv = plsc.load_gather(xbuf, [e16, fv])
                        plsc.store_scatter(obuf, [e16, fv], wr16 * xv)
                return 0
            lax.fori_loop(0, B // 16, grp, 0)
            pltpu.sync_copy(obuf, acc.at[ridx], add=True)
            return 0
        lax.fori_loop(0, nblk, body, 0)
        plsc.subcore_barrier()

        def cp(i, _):
            off = s * rpt + i * B
            pltpu.sync_copy(acc.at[pl.ds(off, B)],
                            out_h.at[c, pl.ds(off, B)])
            return 0
        lax.fori_loop(0, rpt // B, cp, 0)
    return k


_cprop = {wd: _make_prop(wd, EH_PAD, True) for wd in (128, 64, 32, 16)}
_rprop = {wd: _make_prop(wd, E_PAD, False) for wd in (128, 64, 32)}


def _tc(body, out_shape):
    return pl.pallas_call(body, out_shape=out_shape)


def _prep_body(ew_ref, cos_o, sin_o, wsym_o):
    w = ew_ref[...]
    th = (2.0 * jnp.pi * Q) * w
    cos_o[...] = jnp.cos(th)
    sin_o[...] = jnp.sin(th)
    wsym_o[...] = 0.5 * w


def _dinv_body(degp_ref, dm_o, dg_o, sn_o):
    dm = degp_ref[0, 0] + degp_ref[1, 0]
    dg = degp_ref[0, 1] + degp_ref[1, 1] + 1.0
    dm_o[...] = jnp.where(dm > 0, lax.rsqrt(jnp.where(dm > 0, dm, 1.0)), 0.0)
    dgi = lax.rsqrt(dg)
    dg_o[...] = dgi
    sn_o[...] = dgi * dgi


def _pad128(x):
    w = x.shape[1]
    if w == 128:
        return x
    return jnp.pad(x, ((0, 0), (0, 128 - w)))


def _cheb_proj(xr, xi, wcat, b, dout):
    ur = jnp.dot(xr, wcat, preferred_element_type=jnp.float32)
    ui = jnp.dot(xi, wcat, preferred_element_type=jnp.float32)
    s = _pad128(jnp.concatenate(
        [ur[:, :2 * dout], ui[:, :2 * dout]], axis=1))
    a = jnp.concatenate(
        [ur[:, 2 * dout:] - ur[:, dout:2 * dout] + b,
         ui[:, 2 * dout:] - ui[:, dout:2 * dout] + b], axis=1)
    return s, a


def _proj1_body(xr_ref, xi_ref, w_ref, b_ref, s_o, a_o):
    s, a = _cheb_proj(xr_ref[...], xi_ref[...], w_ref[...], b_ref[...], 32)
    s_o[...] = s
    a_o[...] = a


def _mid_body_factory(dout):
    def body(p_ref, g_o):
        p = p_ref[0] + p_ref[1]
        g_o[...] = _pad128(jnp.concatenate(
            [p[:, dout:2 * dout], p[:, 3 * dout:4 * dout]], axis=1))
    return body


def _combine(a_ref, p_ref, q_ref, dout):
    p = p_ref[0] + p_ref[1]
    q = q_ref[0] + q_ref[1]
    a = a_ref[...]
    xr = a[:, :dout] + p[:, :dout] + 2.0 * q[:, :dout]
    xi = a[:, dout:] + p[:, 2 * dout:3 * dout] + 2.0 * q[:, dout:2 * dout]
    return xr, xi


def _proj_next_body_factory(din, dout):
    def body(a_ref, p_ref, q_ref, w_ref, b_ref, s_o, a_o):
        xr, xi = _combine(a_ref, p_ref, q_ref, din)
        s, a = _cheb_proj(xr, xi, w_ref[...], b_ref[...], dout)
        s_o[...] = s
        a_o[...] = a
    return body


def _fin_cheb_body(a_ref, p_ref, q_ref, x1_o):
    xr, xi = _combine(a_ref, p_ref, q_ref, 8)
    x1_o[...] = jnp.concatenate([xr, xi], axis=1)


def _gproj1_body(x_ref, w_ref, sup_o):
    sup_o[...] = jnp.dot(x_ref[...], w_ref[...],
                         preferred_element_type=jnp.float32)


def _gnext_body_factory(din, relu=True):
    def body(gp_ref, sup_ref, sn_ref, bs_ref, bc_ref, ws_ref, wc_ref, sup_o):
        tot = gp_ref[0] + gp_ref[1] + sn_ref[...] * sup_ref[...]
        hs = tot[:, :din] + bs_ref[...]
        hc = tot[:, din:2 * din] + bc_ref[...]
        if relu:
            hs = jnp.maximum(hs, 0.0)
            hc = jnp.maximum(hc, 0.0)
        sup_o[...] = _pad128(jnp.concatenate(
            [jnp.dot(hs, ws_ref[...], preferred_element_type=jnp.float32),
             jnp.dot(hc, wc_ref[...], preferred_element_type=jnp.float32)],
            axis=1))
    return body


def _gfin_body(gp_ref, sup_ref, sn_ref, bs_ref, bc_ref, x2_o, com_o):
    tot = gp_ref[0] + gp_ref[1] + sn_ref[...] * sup_ref[...]
    x2_o[...] = tot[:, :16] + bs_ref[...]
    com_o[...] = tot[:, 16:32] + bc_ref[...]


def _attn_body(x1_ref, x2_ref, xc_ref, a1_ref, ab_ref, a2_ref,
               m1_ref, mb1_ref, m2_ref, mb2_ref, m3_ref, mb3_ref,
               logp_o, beta_o, emb_o):
    x1, x2, xc = x1_ref[...], x2_ref[...], xc_ref[...]
    a1, ab, a2 = a1_ref[...], ab_ref[...], a2_ref[...]

    def score(z):
        h = jnp.tanh(jnp.dot(z, a1, preferred_element_type=jnp.float32) + ab)
        return jnp.dot(h, a2, preferred_element_type=jnp.float32)

    w = jnp.concatenate([score(x1), score(x2), score(xc)], axis=1)
    w = w - jnp.max(w, axis=1, keepdims=True)
    ew = jnp.exp(w)
    beta = ew / jnp.sum(ew, axis=1, keepdims=True)
    beta_o[...] = beta
    emb = (beta[:, 0:1] * x1 + beta[:, 1:2] * x2 + beta[:, 2:3] * xc)
    emb_o[...] = emb
    h = jnp.dot(emb, m1_ref[...], preferred_element_type=jnp.float32) + mb1_ref[...]
    h = jnp.dot(h, m2_ref[...], preferred_element_type=jnp.float32) + mb2_ref[...]
    h = jnp.dot(h, m3_ref[...], preferred_element_type=jnp.float32) + mb3_ref[...]
    h = h - jnp.max(h, axis=1, keepdims=True)
    logp_o[...] = h - jnp.log(jnp.sum(jnp.exp(h), axis=1, keepdims=True))


def _f32(*shape):
    return jax.ShapeDtypeStruct(shape, jnp.float32)


def kernel(x, real, imag, edge_index, edge_weight, params):
    p = params
    src = edge_index[0]
    dst = edge_index[1]

    def padn(a):
        return jnp.pad(a, ((0, N_PAD - N), (0, 0)))

    def pade(a, tot):
        return jnp.pad(a, (0, tot - a.shape[0]))

    xp = padn(x)
    xrp = padn(real)
    xip = padn(imag)
    src_p = pade(src, E_PAD)
    dst_p = pade(dst, E_PAD)
    ew_p = pade(edge_weight, E_PAD)

    cos_e, sin_e, wsym_e = _tc(_prep_body, (_f32(1250, 128),) * 3)(
        edge_weight.reshape(1250, 128))
    cos_p = pade(cos_e.reshape(E), E_PAD)
    sin_p = pade(sin_e.reshape(E), E_PAD)
    wsym_p = pade(wsym_e.reshape(E), E_PAD)

    degp = _sc_degrees(src_p, dst_p, wsym_p, ew_p)
    dm, dg, selfn = _tc(_dinv_body, (_f32(80, 128),) * 3)(
        degp.reshape(NC, 2, 80, 128))
    dm = dm.reshape(N_PAD // 128, 128)
    dg = dg.reshape(N_PAD // 128, 128)
    selfn = selfn.reshape(N_PAD, 1)

    nr_e, ni_e, nin_e, gn_e = _sc_norms(
        src_p, dst_p, wsym_p, ew_p, cos_p, sin_p, dm, dg)

    hrows = pade(jnp.concatenate([src, dst]), EH_PAD)
    hcols = pade(jnp.concatenate([dst, src]), EH_PAD)
    hnr = pade(jnp.concatenate([nr_e[:E], nr_e[:E]]), EH_PAD)
    hni = pade(jnp.concatenate([ni_e[:E], nin_e[:E]]), EH_PAD)

    w1cat = jnp.concatenate(
        [p['cheb1_W'][1], p['cheb1_W'][2], p['cheb1_W'][0]], axis=1)
    w2cat = jnp.concatenate(
        [p['cheb2_W'][1], p['cheb2_W'][2], p['cheb2_W'][0]], axis=1)
    w3cat = jnp.concatenate(
        [p['cheb3_W'][1], p['cheb3_W'][2], p['cheb3_W'][0]], axis=1)

    s1, a1 = _tc(_proj1_body, (_f32(N_PAD, 128), _f32(N_PAD, 64)))(
        xrp, xip, w1cat, p['cheb1_b'])
    p1 = _cprop[128](s1, hrows, hcols, hnr, hni)
    g1 = _tc(_mid_body_factory(32), _f32(N_PAD, 128))(p1)
    q1 = _cprop[64](g1, hrows, hcols, hnr, hni)

    s2, a2 = _tc(_proj_next_body_factory(32, 16),
                 (_f32(N_PAD, 128), _f32(N_PAD, 32)))(
        a1, p1, q1, w2cat, p['cheb2_b'])
    p2 = _cprop[64](s2, hrows, hcols, hnr, hni)
    g2 = _tc(_mid_body_factory(16), _f32(N_PAD, 128))(p2)
    q2 = _cprop[32](g2, hrows, hcols, hnr, hni)

    s3, a3 = _tc(_proj_next_body_factory(16, 8),
                 (_f32(N_PAD, 128), _f32(N_PAD, 16)))(
        a2, p2, q2, w3cat, p['cheb3_b'])
    p3 = _cprop[32](s3, hrows, hcols, hnr, hni)
    g3 = _tc(_mid_body_factory(8), _f32(N_PAD, 128))(p3)
    q3 = _cprop[16](g3, hrows, hcols, hnr, hni)

    x1p = _tc(_fin_cheb_body, _f32(N_PAD, 16))(a3, p3, q3)

    gn_dep = gn_e + q3[0, 0, 0] * 0.0
    wg1 = jnp.concatenate([p['sgcn_W1'], p['cgcn_W1']], axis=1)
    sup1 = _tc(_gproj1_body, _f32(N_PAD, 128))(xp, wg1)
    gp1 = _rprop[128](sup1, dst_p, src_p, gn_dep, gn_dep)
    sup2 = _tc(_gnext_body_factory(64), _f32(N_PAD, 128))(
        gp1, sup1, selfn, p['sgcn_b1'], p['cgcn_b1'],
        p['sgcn_W2'], p['cgcn_W2'])
    gp2 = _rprop[64](sup2, dst_p, src_p, gn_e, gn_e)
    sup3 = _tc(_gnext_body_factory(32), _f32(N_PAD, 128))(
        gp2, sup2, selfn, p['sgcn_b2'], p['cgcn_b2'],
        p['sgcn_W3'], p['cgcn_W3'])
    gp3 = _rprop[32](sup3, dst_p, src_p, gn_e, gn_e)
    x2p, comp = _tc(_gfin_body, (_f32(N_PAD, 16), _f32(N_PAD, 16)))(
        gp3, sup3, selfn, p['sgcn_b3'], p['cgcn_b3'])

    logp_p, beta_p, emb_p = _tc(
        _attn_body, (_f32(N_PAD, 8), _f32(N_PAD, 3), _f32(N_PAD, 16)))(
        x1p, x2p, comp, p['attn_W1'], p['attn_b1'], p['attn_W2'],
        p['mlp_W1'], p['mlp_b1'], p['mlp_W2'], p['mlp_b2'],
        p['mlp_W3'], p['mlp_b3'])

    logp = logp_p[:N]
    beta = beta_p[:N, :, None]
    x1 = x1p[:N]
    com1 = comp[:N]
    x2 = x2p[:N]
    emb = emb_p[:N]
    return (logp, beta, x1, com1, com1, x2, emb)


# speedup vs baseline: 3.7950x; 1.1948x over previous
import functools

import jax
import jax.numpy as jnp
from jax import lax
from jax.experimental import pallas as pl
from jax.experimental.pallas import tpu as pltpu
from jax.experimental.pallas import tpu_sc as plsc

N = 10000
E = 160000
N_PAD = 10240
NC, NS, NW = 2, 16, 32
B = 128
Q = 0.25

E_PAD = 163840
EH_PAD = 327680

_mesh = plsc.VectorSubcoreMesh(
    core_axis_name="c", subcore_axis_name="s", num_cores=NC, num_subcores=NS)


def _wid():
    return lax.axis_index("s") * NC + lax.axis_index("c")


def _zero_fill(buf, rows, width):
    z = jnp.zeros((16,), jnp.float32)

    def body(r, _):
        for f in range(width // 16):
            buf[r, pl.ds(f * 16, 16)] = z
        return 0
    lax.fori_loop(0, rows, body, 0)


def _zero_fill_1d(buf, n):
    z = jnp.zeros((16,), jnp.float32)

    def body(i, _):
        buf[pl.ds(i * 16, 16)] = z
        return 0
    lax.fori_loop(0, n // 16, body, 0)


_NBLK_E = E_PAD // (NW * B)


@functools.partial(
    pl.kernel,
    out_type=jax.ShapeDtypeStruct((NC, 2, N_PAD), jnp.float32),
    mesh=_mesh,
    compiler_params=pltpu.CompilerParams(needs_layout_passes=False),
    scratch_types=[
        pltpu.VMEM((B,), jnp.int32),
        pltpu.VMEM((B,), jnp.int32),
        pltpu.VMEM((B,), jnp.float32),
        pltpu.VMEM((B,), jnp.float32),
        pltpu.VMEM((B,), jnp.float32),
        pltpu.VMEM_SHARED((N_PAD,), jnp.float32),
        pltpu.VMEM_SHARED((N_PAD,), jnp.float32),
    ],
)
def _sc_degrees(src_h, dst_h, wsym_h, ew_h, out_h,
                sidx, didx, wm, wg, zbuf, accm, accg):
    c = lax.axis_index("c")
    s = lax.axis_index("s")
    w = _wid()
    rpt = N_PAD // NS
    _zero_fill_1d(zbuf, B)

    def zb(i, _):
        pltpu.sync_copy(zbuf, accm.at[pl.ds(s * rpt + i * B, B)])
        pltpu.sync_copy(zbuf, accg.at[pl.ds(s * rpt + i * B, B)])
        return 0
    lax.fori_loop(0, rpt // B, zb, 0)
    plsc.subcore_barrier()

    def body(b, _):
        base = (w * _NBLK_E + b) * B
        pltpu.sync_copy(src_h.at[pl.ds(base, B)], sidx)
        pltpu.sync_copy(dst_h.at[pl.ds(base, B)], didx)
        pltpu.sync_copy(wsym_h.at[pl.ds(base, B)], wm)
        pltpu.sync_copy(ew_h.at[pl.ds(base, B)], wg)
        pltpu.sync_copy(wm, accm.at[sidx], add=True)
        pltpu.sync_copy(wm, accm.at[didx], add=True)
        pltpu.sync_copy(wg, accg.at[didx], add=True)
        return 0
    lax.fori_loop(0, _NBLK_E, body, 0)
    plsc.subcore_barrier()

    def cp(i, _):
        off = s * rpt + i * B
        pltpu.sync_copy(accm.at[pl.ds(off, B)], out_h.at[c, 0, pl.ds(off, B)])
        pltpu.sync_copy(accg.at[pl.ds(off, B)], out_h.at[c, 1, pl.ds(off, B)])
        return 0
    lax.fori_loop(0, rpt // B, cp, 0)


@functools.partial(
    pl.kernel,
    out_type=[jax.ShapeDtypeStruct((E_PAD,), jnp.float32) for _ in range(4)],
    mesh=_mesh,
    compiler_params=pltpu.CompilerParams(needs_layout_passes=False),
    scratch_types=[
        pltpu.VMEM((N_PAD // 128, 128), jnp.float32),
        pltpu.VMEM((N_PAD // 128, 128), jnp.float32),
        pltpu.VMEM((B,), jnp.int32),
        pltpu.VMEM((B,), jnp.int32),
        pltpu.VMEM((B,), jnp.float32),
        pltpu.VMEM((B,), jnp.float32),
        pltpu.VMEM((B,), jnp.float32),
        pltpu.VMEM((B,), jnp.float32),
        pltpu.VMEM((B,), jnp.float32),
        pltpu.VMEM((B,), jnp.float32),
        pltpu.VMEM((B,), jnp.float32),
        pltpu.VMEM((B,), jnp.float32),
    ],
)
def _sc_norms(src_h, dst_h, wsym_h, ew_h, cos_h, sin_h, dm_h, dg_h,
              nr_o, ni_o, nin_o, gn_o,
              dmt, dgt, sidx, didx, wm, wg, cth, sth, bnr, bni, bnin, bgn):
    w = _wid()
    pltpu.sync_copy(dm_h, dmt)
    pltpu.sync_copy(dg_h, dgt)

    def body(b, _):
        base = (w * _NBLK_E + b) * B
        pltpu.sync_copy(src_h.at[pl.ds(base, B)], sidx)
        pltpu.sync_copy(dst_h.at[pl.ds(base, B)], didx)
        pltpu.sync_copy(wsym_h.at[pl.ds(base, B)], wm)
        pltpu.sync_copy(ew_h.at[pl.ds(base, B)], wg)
        pltpu.sync_copy(cos_h.at[pl.ds(base, B)], cth)
        pltpu.sync_copy(sin_h.at[pl.ds(base, B)], sth)

        def grp(g, _):
            o = g * 16
            s16 = sidx[pl.ds(o, 16)]
            d16 = didx[pl.ds(o, 16)]
            srow = lax.shift_right_logical(s16, 7)
            scol = lax.bitwise_and(s16, 127)
            drow = lax.shift_right_logical(d16, 7)
            dcol = lax.bitwise_and(d16, 127)
            dms = plsc.load_gather(dmt, [srow, scol])
            dmd = plsc.load_gather(dmt, [drow, dcol])
            dgs = plsc.load_gather(dgt, [srow, scol])
            dgd = plsc.load_gather(dgt, [drow, dcol])
            nrm = dms * wm[pl.ds(o, 16)] * dmd
            nr = -nrm * cth[pl.ds(o, 16)]
            ni = -nrm * sth[pl.ds(o, 16)]
            bnr[pl.ds(o, 16)] = nr
            bni[pl.ds(o, 16)] = ni
            bnin[pl.ds(o, 16)] = -ni
            bgn[pl.ds(o, 16)] = dgd * wg[pl.ds(o, 16)] * dgs
            return 0
        lax.fori_loop(0, B // 16, grp, 0)
        pltpu.sync_copy(bnr, nr_o.at[pl.ds(base, B)])
        pltpu.sync_copy(bni, ni_o.at[pl.ds(base, B)])
        pltpu.sync_copy(bnin, nin_o.at[pl.ds(base, B)])
        pltpu.sync_copy(bgn, gn_o.at[pl.ds(base, B)])
        return 0
    lax.fori_loop(0, _NBLK_E, body, 0)


def _make_prop(real_w, n_edges, is_complex):
    nblk = n_edges // (NW * B)
    dc = real_w // 2
    W128 = real_w

    @functools.partial(
        pl.kernel,
        out_type=jax.ShapeDtypeStruct((NC, N_PAD, W128), jnp.float32),
        mesh=_mesh,
        compiler_params=pltpu.CompilerParams(
            needs_layout_passes=False, use_tc_tiling_on_sc=False),
        scratch_types=[
            pltpu.VMEM((B,), jnp.int32),
            pltpu.VMEM((B,), jnp.int32),
            pltpu.VMEM((B,), jnp.float32),
            pltpu.VMEM((B,), jnp.float32),
            pltpu.VMEM((B, W128), jnp.float32),
            pltpu.VMEM((B, W128), jnp.float32),
            pltpu.VMEM_SHARED((N_PAD, W128), jnp.float32),
            pltpu.SemaphoreType.DMA,
        ],
    )
    def k(xf_h, rows_h, cols_h, wr_h, wi_h, out_h,
          cidx, ridx, wrv, wiv, xbuf, obuf, acc, sem):
        c = lax.axis_index("c")
        s = lax.axis_index("s")
        w = _wid()
        rpt = N_PAD // NS
        _zero_fill(obuf, B, W128)

        def zb(i, _):
            pltpu.sync_copy(obuf, acc.at[pl.ds(s * rpt + i * B, B)])
            return 0
        lax.fori_loop(0, rpt // B, zb, 0)
        plsc.subcore_barrier()

        def body(b, _):
            base = (w * nblk + b) * B
            pltpu.sync_copy(cols_h.at[pl.ds(base, B)], cidx)
            pltpu.sync_copy(rows_h.at[pl.ds(base, B)], ridx)
            pltpu.sync_copy(wr_h.at[pl.ds(base, B)], wrv)
            if is_complex:
                pltpu.sync_copy(wi_h.at[pl.ds(base, B)], wiv)
            pltpu.async_copy(xf_h.at[cidx], xbuf, sem).wait()

            def grp(g, _):
                e16 = g * 16 + lax.iota(jnp.int32, 16)
                wr16 = wrv[pl.ds(g * 16, 16)]
                if is_complex:
                    wi16 = wiv[pl.ds(g * 16, 16)]
                    for f in range(dc):
                        fv = jnp.full((16,), f, jnp.int32)
                        fv2 = jnp.full((16,), f + dc, jnp.int32)
                        xr = plsc.load_gather(xbuf, [e16, fv])
                        xi = plsc.load_gather(xbuf, [e16, fv2])
                        plsc.store_scatter(obuf, [e16, fv],
                                           wr16 * xr - wi16 * xi)
                        plsc.store_scatter(obuf, [e16, fv2],
                                           wi16 * xr + wr16 * xi)
                else:
                    for f in range(real_w):
                        fv = jnp.full((16,), f, jnp.int32)
                        xv = plsc.load_gather(xbuf, [e16, fv])
                        plsc.store_scatter(obuf, [e16, fv], wr16 * xv)
                return 0
            lax.fori_loop(0, B // 16, grp, 0)
            pltpu.sync_copy(obuf, acc.at[ridx], add=True)
            return 0
        lax.fori_loop(0, nblk, body, 0)
        plsc.subcore_barrier()

        def cp(i, _):
            off = s * rpt + i * B
            pltpu.sync_copy(acc.at[pl.ds(off, B)],
                            out_h.at[c, pl.ds(off, B)])
            return 0
        lax.fori_loop(0, rpt // B, cp, 0)
    return k


_cprop = {wd: _make_prop(wd, EH_PAD, True) for wd in (128, 64, 32, 16)}
_rprop = {wd: _make_prop(wd, E_PAD, False) for wd in (128, 64, 32)}


def _tc(body, out_shape):
    return pl.pallas_call(body, out_shape=out_shape)


def _prep_body(ew_ref, cos_o, sin_o, wsym_o):
    w = ew_ref[...]
    th = (2.0 * jnp.pi * Q) * w
    cos_o[...] = jnp.cos(th)
    sin_o[...] = jnp.sin(th)
    wsym_o[...] = 0.5 * w


def _dinv_body(degp_ref, dm_o, dg_o, sn_o):
    dm = degp_ref[0, 0] + degp_ref[1, 0]
    dg = degp_ref[0, 1] + degp_ref[1, 1] + 1.0
    dm_o[...] = jnp.where(dm > 0, lax.rsqrt(jnp.where(dm > 0, dm, 1.0)), 0.0)
    dgi = lax.rsqrt(dg)
    dg_o[...] = dgi
    sn_o[...] = dgi * dgi


def _pad128(x):
    w = x.shape[1]
    if w == 128:
        return x
    return jnp.pad(x, ((0, 0), (0, 128 - w)))


def _cheb_proj(xr, xi, wcat, b, dout):
    ur = jnp.dot(xr, wcat, preferred_element_type=jnp.float32)
    ui = jnp.dot(xi, wcat, preferred_element_type=jnp.float32)
    s = jnp.concatenate(
        [ur[:, :2 * dout], ui[:, :2 * dout]], axis=1)
    a = jnp.concatenate(
        [ur[:, 2 * dout:] - ur[:, dout:2 * dout] + b,
         ui[:, 2 * dout:] - ui[:, dout:2 * dout] + b], axis=1)
    return s, a


def _proj1_body(xr_ref, xi_ref, w_ref, b_ref, s_o, a_o):
    s, a = _cheb_proj(xr_ref[...], xi_ref[...], w_ref[...], b_ref[...], 32)
    s_o[...] = s
    a_o[...] = a


def _mid_body_factory(dout):
    def body(p_ref, g_o):
        p = p_ref[0] + p_ref[1]
        g_o[...] = jnp.concatenate(
            [p[:, dout:2 * dout], p[:, 3 * dout:4 * dout]], axis=1)
    return body


def _combine(a_ref, p_ref, q_ref, dout):
    p = p_ref[0] + p_ref[1]
    q = q_ref[0] + q_ref[1]
    a = a_ref[...]
    xr = a[:, :dout] + p[:, :dout] + 2.0 * q[:, :dout]
    xi = a[:, dout:] + p[:, 2 * dout:3 * dout] + 2.0 * q[:, dout:2 * dout]
    return xr, xi


def _proj_next_body_factory(din, dout):
    def body(a_ref, p_ref, q_ref, w_ref, b_ref, s_o, a_o):
        xr, xi = _combine(a_ref, p_ref, q_ref, din)
        s, a = _cheb_proj(xr, xi, w_ref[...], b_ref[...], dout)
        s_o[...] = s
        a_o[...] = a
    return body


def _fin_cheb_body(a_ref, p_ref, q_ref, x1_o):
    xr, xi = _combine(a_ref, p_ref, q_ref, 8)
    x1_o[...] = jnp.concatenate([xr, xi], axis=1)


def _gproj1_body(x_ref, w_ref, sup_o):
    sup_o[...] = jnp.dot(x_ref[...], w_ref[...],
                         preferred_element_type=jnp.float32)


def _gnext_body_factory(din, relu=True):
    def body(gp_ref, sup_ref, sn_ref, bs_ref, bc_ref, ws_ref, wc_ref, sup_o):
        tot = gp_ref[0] + gp_ref[1] + sn_ref[...] * sup_ref[...]
        hs = tot[:, :din] + bs_ref[...]
        hc = tot[:, din:2 * din] + bc_ref[...]
        if relu:
            hs = jnp.maximum(hs, 0.0)
            hc = jnp.maximum(hc, 0.0)
        sup_o[...] = jnp.concatenate(
            [jnp.dot(hs, ws_ref[...], preferred_element_type=jnp.float32),
             jnp.dot(hc, wc_ref[...], preferred_element_type=jnp.float32)],
            axis=1)
    return body


def _gfin_body(gp_ref, sup_ref, sn_ref, bs_ref, bc_ref, x2_o, com_o):
    tot = gp_ref[0] + gp_ref[1] + sn_ref[...] * sup_ref[...]
    x2_o[...] = tot[:, :16] + bs_ref[...]
    com_o[...] = tot[:, 16:32] + bc_ref[...]


def _attn_body(x1_ref, x2_ref, xc_ref, a1_ref, ab_ref, a2_ref,
               m1_ref, mb1_ref, m2_ref, mb2_ref, m3_ref, mb3_ref,
               logp_o, beta_o, emb_o):
    x1, x2, xc = x1_ref[...], x2_ref[...], xc_ref[...]
    a1, ab, a2 = a1_ref[...], ab_ref[...], a2_ref[...]

    def score(z):
        h = jnp.tanh(jnp.dot(z, a1, preferred_element_type=jnp.float32) + ab)
        return jnp.dot(h, a2, preferred_element_type=jnp.float32)

    w = jnp.concatenate([score(x1), score(x2), score(xc)], axis=1)
    w = w - jnp.max(w, axis=1, keepdims=True)
    ew = jnp.exp(w)
    beta = ew / jnp.sum(ew, axis=1, keepdims=True)
    beta_o[...] = beta
    emb = (beta[:, 0:1] * x1 + beta[:, 1:2] * x2 + beta[:, 2:3] * xc)
    emb_o[...] = emb
    h = jnp.dot(emb, m1_ref[...], preferred_element_type=jnp.float32) + mb1_ref[...]
    h = jnp.dot(h, m2_ref[...], preferred_element_type=jnp.float32) + mb2_ref[...]
    h = jnp.dot(h, m3_ref[...], preferred_element_type=jnp.float32) + mb3_ref[...]
    h = h - jnp.max(h, axis=1, keepdims=True)
    logp_o[...] = h - jnp.log(jnp.sum(jnp.exp(h), axis=1, keepdims=True))


def _f32(*shape):
    return jax.ShapeDtypeStruct(shape, jnp.float32)


def kernel(x, real, imag, edge_index, edge_weight, params):
    p = params
    src = edge_index[0]
    dst = edge_index[1]

    def padn(a):
        return jnp.pad(a, ((0, N_PAD - N), (0, 0)))

    def pade(a, tot):
        return jnp.pad(a, (0, tot - a.shape[0]))

    xp = padn(x)
    xrp = padn(real)
    xip = padn(imag)
    src_p = pade(src, E_PAD)
    dst_p = pade(dst, E_PAD)
    ew_p = pade(edge_weight, E_PAD)

    cos_e, sin_e, wsym_e = _tc(_prep_body, (_f32(1250, 128),) * 3)(
        edge_weight.reshape(1250, 128))
    cos_p = pade(cos_e.reshape(E), E_PAD)
    sin_p = pade(sin_e.reshape(E), E_PAD)
    wsym_p = pade(wsym_e.reshape(E), E_PAD)

    degp = _sc_degrees(src_p, dst_p, wsym_p, ew_p)
    dm, dg, selfn = _tc(_dinv_body, (_f32(80, 128),) * 3)(
        degp.reshape(NC, 2, 80, 128))
    dm = dm.reshape(N_PAD // 128, 128)
    dg = dg.reshape(N_PAD // 128, 128)
    selfn = selfn.reshape(N_PAD, 1)

    nr_e, ni_e, nin_e, gn_e = _sc_norms(
        src_p, dst_p, wsym_p, ew_p, cos_p, sin_p, dm, dg)

    hrows = pade(jnp.concatenate([src, dst]), EH_PAD)
    hcols = pade(jnp.concatenate([dst, src]), EH_PAD)
    hnr = pade(jnp.concatenate([nr_e[:E], nr_e[:E]]), EH_PAD)
    hni = pade(jnp.concatenate([ni_e[:E], nin_e[:E]]), EH_PAD)

    w1cat = jnp.concatenate(
        [p['cheb1_W'][1], p['cheb1_W'][2], p['cheb1_W'][0]], axis=1)
    w2cat = jnp.concatenate(
        [p['cheb2_W'][1], p['cheb2_W'][2], p['cheb2_W'][0]], axis=1)
    w3cat = jnp.concatenate(
        [p['cheb3_W'][1], p['cheb3_W'][2], p['cheb3_W'][0]], axis=1)

    s1, a1 = _tc(_proj1_body, (_f32(N_PAD, 128), _f32(N_PAD, 64)))(
        xrp, xip, w1cat, p['cheb1_b'])
    p1 = _cprop[128](s1, hrows, hcols, hnr, hni)
    g1 = _tc(_mid_body_factory(32), _f32(N_PAD, 64))(p1)
    q1 = _cprop[64](g1, hrows, hcols, hnr, hni)

    s2, a2 = _tc(_proj_next_body_factory(32, 16),
                 (_f32(N_PAD, 64), _f32(N_PAD, 32)))(
        a1, p1, q1, w2cat, p['cheb2_b'])
    p2 = _cprop[64](s2, hrows, hcols, hnr, hni)
    g2 = _tc(_mid_body_factory(16), _f32(N_PAD, 32))(p2)
    q2 = _cprop[32](g2, hrows, hcols, hnr, hni)

    s3, a3 = _tc(_proj_next_body_factory(16, 8),
                 (_f32(N_PAD, 32), _f32(N_PAD, 16)))(
        a2, p2, q2, w3cat, p['cheb3_b'])
    p3 = _cprop[32](s3, hrows, hcols, hnr, hni)
    g3 = _tc(_mid_body_factory(8), _f32(N_PAD, 16))(p3)
    q3 = _cprop[16](g3, hrows, hcols, hnr, hni)

    x1p = _tc(_fin_cheb_body, _f32(N_PAD, 16))(a3, p3, q3)

    gn_dep = gn_e + q3[0, 0, 0] * 0.0
    wg1 = jnp.concatenate([p['sgcn_W1'], p['cgcn_W1']], axis=1)
    sup1 = _tc(_gproj1_body, _f32(N_PAD, 128))(xp, wg1)
    gp1 = _rprop[128](sup1, dst_p, src_p, gn_dep, gn_dep)
    sup2 = _tc(_gnext_body_factory(64), _f32(N_PAD, 64))(
        gp1, sup1, selfn, p['sgcn_b1'], p['cgcn_b1'],
        p['sgcn_W2'], p['cgcn_W2'])
    gp2 = _rprop[64](sup2, dst_p, src_p, gn_e, gn_e)
    sup3 = _tc(_gnext_body_factory(32), _f32(N_PAD, 32))(
        gp2, sup2, selfn, p['sgcn_b2'], p['cgcn_b2'],
        p['sgcn_W3'], p['cgcn_W3'])
    gp3 = _rprop[32](sup3, dst_p, src_p, gn_e, gn_e)
    x2p, comp = _tc(_gfin_body, (_f32(N_PAD, 16), _f32(N_PAD, 16)))(
        gp3, sup3, selfn, p['sgcn_b3'], p['cgcn_b3'])

    logp_p, beta_p, emb_p = _tc(
        _attn_body, (_f32(N_PAD, 8), _f32(N_PAD, 3), _f32(N_PAD, 16)))(
        x1p, x2p, comp, p['attn_W1'], p['attn_b1'], p['attn_W2'],
        p['mlp_W1'], p['mlp_b1'], p['mlp_W2'], p['mlp_b2'],
        p['mlp_W3'], p['mlp_b3'])

    logp = logp_p[:N]
    beta = beta_p[:N, :, None]
    x1 = x1p[:N]
    com1 = comp[:N]
    x2 = x2p[:N]
    emb = emb_p[:N]
    return (logp, beta, x1, com1, com1, x2, emb)


# speedup vs baseline: 7.8220x; 2.0612x over previous
import functools

import jax
import jax.numpy as jnp
from jax import lax
from jax.experimental import pallas as pl
from jax.experimental.pallas import tpu as pltpu
from jax.experimental.pallas import tpu_sc as plsc

N = 10000
E = 160000
N_PAD = 10240
NC, NS, NW = 2, 16, 32
B = 128
Q = 0.25

E_PAD = 163840
EH_PAD = 327680
PF_PAD = 256

_mesh = plsc.VectorSubcoreMesh(
    core_axis_name="c", subcore_axis_name="s", num_cores=NC, num_subcores=NS)


def _wid():
    return lax.axis_index("s") * NC + lax.axis_index("c")


def _zero_fill(buf, rows, width):
    z = jnp.zeros((16,), jnp.float32)

    def body(r, _):
        for f in range(width // 16):
            buf[r, pl.ds(f * 16, 16)] = z
        return 0
    lax.fori_loop(0, rows, body, 0)


def _zero_fill_1d(buf, n):
    z = jnp.zeros((16,), jnp.float32)

    def body(i, _):
        buf[pl.ds(i * 16, 16)] = z
        return 0
    lax.fori_loop(0, n // 16, body, 0)


_NBLK_E = E_PAD // (NW * B)


@functools.partial(
    pl.kernel,
    out_type=jax.ShapeDtypeStruct((NC, 2, N_PAD), jnp.float32),
    mesh=_mesh,
    compiler_params=pltpu.CompilerParams(needs_layout_passes=False),
    scratch_types=[
        pltpu.VMEM((B,), jnp.int32),
        pltpu.VMEM((B,), jnp.int32),
        pltpu.VMEM((B,), jnp.float32),
        pltpu.VMEM((B,), jnp.float32),
        pltpu.VMEM((B,), jnp.float32),
        pltpu.VMEM_SHARED((N_PAD,), jnp.float32),
        pltpu.VMEM_SHARED((N_PAD,), jnp.float32),
    ],
)
def _sc_degrees(src_h, dst_h, wsym_h, ew_h, out_h,
                sidx, didx, wm, wg, zbuf, accm, accg):
    c = lax.axis_index("c")
    s = lax.axis_index("s")
    w = _wid()
    rpt = N_PAD // NS
    _zero_fill_1d(zbuf, B)

    def zb(i, _):
        pltpu.sync_copy(zbuf, accm.at[pl.ds(s * rpt + i * B, B)])
        pltpu.sync_copy(zbuf, accg.at[pl.ds(s * rpt + i * B, B)])
        return 0
    lax.fori_loop(0, rpt // B, zb, 0)
    plsc.subcore_barrier()

    def body(b, _):
        base = (w * _NBLK_E + b) * B
        pltpu.sync_copy(src_h.at[pl.ds(base, B)], sidx)
        pltpu.sync_copy(dst_h.at[pl.ds(base, B)], didx)
        pltpu.sync_copy(wsym_h.at[pl.ds(base, B)], wm)
        pltpu.sync_copy(ew_h.at[pl.ds(base, B)], wg)
        pltpu.sync_copy(wm, accm.at[sidx], add=True)
        pltpu.sync_copy(wm, accm.at[didx], add=True)
        pltpu.sync_copy(wg, accg.at[didx], add=True)
        return 0
    lax.fori_loop(0, _NBLK_E, body, 0)
    plsc.subcore_barrier()

    def cp(i, _):
        off = s * rpt + i * B
        pltpu.sync_copy(accm.at[pl.ds(off, B)], out_h.at[c, 0, pl.ds(off, B)])
        pltpu.sync_copy(accg.at[pl.ds(off, B)], out_h.at[c, 1, pl.ds(off, B)])
        return 0
    lax.fori_loop(0, rpt // B, cp, 0)


@functools.partial(
    pl.kernel,
    out_type=[jax.ShapeDtypeStruct((E_PAD,), jnp.float32) for _ in range(4)],
    mesh=_mesh,
    compiler_params=pltpu.CompilerParams(needs_layout_passes=False),
    scratch_types=[
        pltpu.VMEM((N_PAD // 128, 128), jnp.float32),
        pltpu.VMEM((N_PAD // 128, 128), jnp.float32),
        pltpu.VMEM((B,), jnp.int32),
        pltpu.VMEM((B,), jnp.int32),
        pltpu.VMEM((B,), jnp.float32),
        pltpu.VMEM((B,), jnp.float32),
        pltpu.VMEM((B,), jnp.float32),
        pltpu.VMEM((B,), jnp.float32),
        pltpu.VMEM((B,), jnp.float32),
        pltpu.VMEM((B,), jnp.float32),
        pltpu.VMEM((B,), jnp.float32),
        pltpu.VMEM((B,), jnp.float32),
    ],
)
def _sc_norms(src_h, dst_h, wsym_h, ew_h, cos_h, sin_h, dm_h, dg_h,
              nr_o, ni_o, nin_o, gn_o,
              dmt, dgt, sidx, didx, wm, wg, cth, sth, bnr, bni, bnin, bgn):
    w = _wid()
    pltpu.sync_copy(dm_h, dmt)
    pltpu.sync_copy(dg_h, dgt)

    def body(b, _):
        base = (w * _NBLK_E + b) * B
        pltpu.sync_copy(src_h.at[pl.ds(base, B)], sidx)
        pltpu.sync_copy(dst_h.at[pl.ds(base, B)], didx)
        pltpu.sync_copy(wsym_h.at[pl.ds(base, B)], wm)
        pltpu.sync_copy(ew_h.at[pl.ds(base, B)], wg)
        pltpu.sync_copy(cos_h.at[pl.ds(base, B)], cth)
        pltpu.sync_copy(sin_h.at[pl.ds(base, B)], sth)

        def grp(g, _):
            o = g * 16
            s16 = sidx[pl.ds(o, 16)]
            d16 = didx[pl.ds(o, 16)]
            srow = lax.shift_right_logical(s16, 7)
            scol = lax.bitwise_and(s16, 127)
            drow = lax.shift_right_logical(d16, 7)
            dcol = lax.bitwise_and(d16, 127)
            dms = plsc.load_gather(dmt, [srow, scol])
            dmd = plsc.load_gather(dmt, [drow, dcol])
            dgs = plsc.load_gather(dgt, [srow, scol])
            dgd = plsc.load_gather(dgt, [drow, dcol])
            nrm = dms * wm[pl.ds(o, 16)] * dmd
            nr = -nrm * cth[pl.ds(o, 16)]
            ni = -nrm * sth[pl.ds(o, 16)]
            bnr[pl.ds(o, 16)] = nr
            bni[pl.ds(o, 16)] = ni
            bnin[pl.ds(o, 16)] = -ni
            bgn[pl.ds(o, 16)] = dgd * wg[pl.ds(o, 16)] * dgs
            return 0
        lax.fori_loop(0, B // 16, grp, 0)
        pltpu.sync_copy(bnr, nr_o.at[pl.ds(base, B)])
        pltpu.sync_copy(bni, ni_o.at[pl.ds(base, B)])
        pltpu.sync_copy(bnin, nin_o.at[pl.ds(base, B)])
        pltpu.sync_copy(bgn, gn_o.at[pl.ds(base, B)])
        return 0
    lax.fori_loop(0, _NBLK_E, body, 0)


def _make_prop(real_w, n_edges, is_complex):
    Bp = 64 if real_w == 128 else 128
    nblk = n_edges // (NW * Bp)
    assert nblk % 4 == 0
    dc = real_w // 2

    idx_scratch = [pltpu.VMEM((Bp,), jnp.int32) for _ in range(8)]
    w_scratch = [pltpu.VMEM((Bp,), jnp.float32) for _ in range(8)]
    buf_scratch = [pltpu.VMEM((Bp, real_w), jnp.float32) for _ in range(4)]
    sem_scratch = [pltpu.SemaphoreType.DMA for _ in range(8)]

    @functools.partial(
        pl.kernel,
        out_type=jax.ShapeDtypeStruct((NC, N_PAD, real_w), jnp.float32),
        mesh=_mesh,
        compiler_params=pltpu.CompilerParams(
            needs_layout_passes=False, use_tc_tiling_on_sc=False),
        scratch_types=[*idx_scratch, *w_scratch, *buf_scratch,
                       pltpu.VMEM_SHARED((N_PAD, real_w), jnp.float32),
                       *sem_scratch],
    )
    def k(xf_h, rows_h, cols_h, wr_h, wi_h, out_h,
          c0, c1, c2, c3, r0, r1, r2, r3,
          wr0, wr1, wr2, wr3, wi0, wi1, wi2, wi3,
          xb0, xb1, ob0, ob1, acc,
          l0, l1, l2, l3, g0, g1, s0, s1):
        cidx = [c0, c1, c2, c3]
        ridx = [r0, r1, r2, r3]
        wrv = [wr0, wr1, wr2, wr3]
        wiv = [wi0, wi1, wi2, wi3]
        xb = [xb0, xb1]
        ob = [ob0, ob1]
        lsem = [l0, l1, l2, l3]
        gsem = [g0, g1]
        ssem = [s0, s1]
        c = lax.axis_index("c")
        s = lax.axis_index("s")
        w = _wid()
        rpt = N_PAD // NS

        def lin_pairs(i, base):
            prs = [(cols_h.at[pl.ds(base, Bp)], cidx[i]),
                   (rows_h.at[pl.ds(base, Bp)], ridx[i]),
                   (wr_h.at[pl.ds(base, Bp)], wrv[i])]
            if is_complex:
                prs.append((wi_h.at[pl.ds(base, Bp)], wiv[i]))
            return prs

        def lin_start(i, base):
            for src, dst in lin_pairs(i, base):
                pltpu.async_copy(src, dst, lsem[i])

        def lin_wait(i, base):
            for src, dst in lin_pairs(i, base):
                pltpu.make_async_copy(src, dst, lsem[i]).wait()

        def g_start(d, i):
            pltpu.async_copy(xf_h.at[cidx[i]], xb[d], gsem[d])

        def g_wait(d, i):
            pltpu.make_async_copy(xf_h.at[cidx[i]], xb[d], gsem[d]).wait()

        def s_start(d, i):
            pltpu.async_copy(ob[d], acc.at[ridx[i]], ssem[d], add=True)

        def s_wait(d, i):
            pltpu.make_async_copy(ob[d], acc.at[ridx[i]], ssem[d]).wait()

        def compute(d, i):
            CH = 8

            def grp(g, _):
                e16 = g * 16 + lax.iota(jnp.int32, 16)
                wr16 = wrv[i][pl.ds(g * 16, 16)]
                if is_complex:
                    wi16 = wiv[i][pl.ds(g * 16, 16)]
                    for f0 in range(0, dc, CH):
                        nf = min(CH, dc - f0)
                        fvs = [jnp.full((16,), f0 + f, jnp.int32)
                               for f in range(nf)]
                        fvs2 = [jnp.full((16,), f0 + f + dc, jnp.int32)
                                for f in range(nf)]
                        xrs = [plsc.load_gather(xb[d], [e16, fv])
                               for fv in fvs]
                        xis = [plsc.load_gather(xb[d], [e16, fv])
                               for fv in fvs2]
                        for f in range(nf):
                            plsc.store_scatter(
                                ob[d], [e16, fvs[f]],
                                wr16 * xrs[f] - wi16 * xis[f])
                        for f in range(nf):
                            plsc.store_scatter(
                                ob[d], [e16, fvs2[f]],
                                wi16 * xrs[f] + wr16 * xis[f])
                else:
                    for f0 in range(0, real_w, 2 * CH):
                        nf = min(2 * CH, real_w - f0)
                        fvs = [jnp.full((16,), f0 + f, jnp.int32)
                               for f in range(nf)]
                        xvs = [plsc.load_gather(xb[d], [e16, fv])
                               for fv in fvs]
                        for f in range(nf):
                            plsc.store_scatter(ob[d], [e16, fvs[f]],
                                               wr16 * xvs[f])
                return 0
            lax.fori_loop(0, Bp // 16, grp, 0)

        _zero_fill(ob0, Bp, real_w)

        def zb(i, _):
            pltpu.sync_copy(ob0, acc.at[pl.ds(s * rpt + i * Bp, Bp)])
            return 0
        lax.fori_loop(0, rpt // Bp, zb, 0)
        plsc.subcore_barrier()

        base0 = w * nblk * Bp

        lin_start(0, base0)
        lin_start(1, base0 + Bp)
        lin_wait(0, base0)
        g_start(0, 0)

        def body(tt, _):
            for j in range(4):
                bb = tt * 4 + j
                base = base0 + bb * Bp
                if j < 2:
                    @pl.when(tt > 0)
                    def _():
                        s_wait(j % 2, (j + 2) % 4)
                else:
                    s_wait(j % 2, (j + 2) % 4)
                lin_start((j + 2) % 4, base + 2 * Bp)
                lin_wait((j + 1) % 4, base + Bp)
                g_start((j + 1) % 2, (j + 1) % 4)
                g_wait(j % 2, j % 4)
                compute(j % 2, j % 4)
                s_start(j % 2, j % 4)
            return 0
        lax.fori_loop(0, nblk // 4, body, 0)

        s_wait(0, 2)
        s_wait(1, 3)
        g_wait(0, 0)
        lin_wait(1, base0 + (nblk + 1) * Bp)
        plsc.subcore_barrier()

        def cp(i, _):
            off = s * rpt + i * Bp
            pltpu.sync_copy(acc.at[pl.ds(off, Bp)],
                            out_h.at[c, pl.ds(off, Bp)])
            return 0
        lax.fori_loop(0, rpt // Bp, cp, 0)
    return k


_cprop = {wd: _make_prop(wd, EH_PAD, True) for wd in (128, 64, 32, 16)}
_rprop = {wd: _make_prop(wd, E_PAD, False) for wd in (128, 64, 32)}


def _tc(body, out_shape):
    return pl.pallas_call(body, out_shape=out_shape)


def _prep_body(ew_ref, cos_o, sin_o, wsym_o):
    w = ew_ref[...]
    th = (2.0 * jnp.pi * Q) * w
    cos_o[...] = jnp.cos(th)
    sin_o[...] = jnp.sin(th)
    wsym_o[...] = 0.5 * w


def _dinv_body(degp_ref, dm_o, dg_o, sn_o):
    dm = degp_ref[0, 0] + degp_ref[1, 0]
    dg = degp_ref[0, 1] + degp_ref[1, 1] + 1.0
    dm_o[...] = jnp.where(dm > 0, lax.rsqrt(jnp.where(dm > 0, dm, 1.0)), 0.0)
    dgi = lax.rsqrt(dg)
    dg_o[...] = dgi
    sn_o[...] = dgi * dgi


def _pad128(x):
    w = x.shape[1]
    if w == 128:
        return x
    return jnp.pad(x, ((0, 0), (0, 128 - w)))


def _cheb_proj(xr, xi, wcat, b, dout):
    ur = jnp.dot(xr, wcat, preferred_element_type=jnp.float32)
    ui = jnp.dot(xi, wcat, preferred_element_type=jnp.float32)
    s = jnp.concatenate(
        [ur[:, :2 * dout], ui[:, :2 * dout]], axis=1)
    a = jnp.concatenate(
        [ur[:, 2 * dout:] - ur[:, dout:2 * dout] + b,
         ui[:, 2 * dout:] - ui[:, dout:2 * dout] + b], axis=1)
    return s, a


def _proj1_body(xr_ref, xi_ref, w_ref, b_ref, s_o, a_o):
    s, a = _cheb_proj(xr_ref[...], xi_ref[...], w_ref[...], b_ref[...], 32)
    s_o[...] = s
    a_o[...] = a


def _mid_body_factory(dout):
    def body(p_ref, g_o):
        p = p_ref[0] + p_ref[1]
        g_o[...] = jnp.concatenate(
            [p[:, dout:2 * dout], p[:, 3 * dout:4 * dout]], axis=1)
    return body


def _combine(a_ref, p_ref, q_ref, dout):
    p = p_ref[0] + p_ref[1]
    q = q_ref[0] + q_ref[1]
    a = a_ref[...]
    xr = a[:, :dout] + p[:, :dout] + 2.0 * q[:, :dout]
    xi = a[:, dout:] + p[:, 2 * dout:3 * dout] + 2.0 * q[:, dout:2 * dout]
    return xr, xi


def _proj_next_body_factory(din, dout):
    def body(a_ref, p_ref, q_ref, w_ref, b_ref, s_o, a_o):
        xr, xi = _combine(a_ref, p_ref, q_ref, din)
        s, a = _cheb_proj(xr, xi, w_ref[...], b_ref[...], dout)
        s_o[...] = s
        a_o[...] = a
    return body


def _fin_cheb_body(a_ref, p_ref, q_ref, x1_o):
    xr, xi = _combine(a_ref, p_ref, q_ref, 8)
    x1_o[...] = jnp.concatenate([xr, xi], axis=1)


def _gproj1_body(x_ref, w_ref, sup_o):
    sup_o[...] = jnp.dot(x_ref[...], w_ref[...],
                         preferred_element_type=jnp.float32)


def _gnext_body_factory(din, relu=True):
    def body(gp_ref, sup_ref, sn_ref, bs_ref, bc_ref, ws_ref, wc_ref, sup_o):
        tot = gp_ref[0] + gp_ref[1] + sn_ref[...] * sup_ref[...]
        hs = tot[:, :din] + bs_ref[...]
        hc = tot[:, din:2 * din] + bc_ref[...]
        if relu:
            hs = jnp.maximum(hs, 0.0)
            hc = jnp.maximum(hc, 0.0)
        sup_o[...] = jnp.concatenate(
            [jnp.dot(hs, ws_ref[...], preferred_element_type=jnp.float32),
             jnp.dot(hc, wc_ref[...], preferred_element_type=jnp.float32)],
            axis=1)
    return body


def _gfin_body(gp_ref, sup_ref, sn_ref, bs_ref, bc_ref, x2_o, com_o):
    tot = gp_ref[0] + gp_ref[1] + sn_ref[...] * sup_ref[...]
    x2_o[...] = tot[:, :16] + bs_ref[...]
    com_o[...] = tot[:, 16:32] + bc_ref[...]


def _attn_body(x1_ref, x2_ref, xc_ref, a1_ref, ab_ref, a2_ref,
               m1_ref, mb1_ref, m2_ref, mb2_ref, m3_ref, mb3_ref,
               logp_o, beta_o, emb_o):
    x1, x2, xc = x1_ref[...], x2_ref[...], xc_ref[...]
    a1, ab, a2 = a1_ref[...], ab_ref[...], a2_ref[...]

    def score(z):
        h = jnp.tanh(jnp.dot(z, a1, preferred_element_type=jnp.float32) + ab)
        return jnp.dot(h, a2, preferred_element_type=jnp.float32)

    w = jnp.concatenate([score(x1), score(x2), score(xc)], axis=1)
    w = w - jnp.max(w, axis=1, keepdims=True)
    ew = jnp.exp(w)
    beta = ew / jnp.sum(ew, axis=1, keepdims=True)
    beta_o[...] = beta
    emb = (beta[:, 0:1] * x1 + beta[:, 1:2] * x2 + beta[:, 2:3] * xc)
    emb_o[...] = emb
    h = jnp.dot(emb, m1_ref[...], preferred_element_type=jnp.float32) + mb1_ref[...]
    h = jnp.dot(h, m2_ref[...], preferred_element_type=jnp.float32) + mb2_ref[...]
    h = jnp.dot(h, m3_ref[...], preferred_element_type=jnp.float32) + mb3_ref[...]
    h = h - jnp.max(h, axis=1, keepdims=True)
    logp_o[...] = h - jnp.log(jnp.sum(jnp.exp(h), axis=1, keepdims=True))


def _f32(*shape):
    return jax.ShapeDtypeStruct(shape, jnp.float32)


def kernel(x, real, imag, edge_index, edge_weight, params):
    p = params
    src = edge_index[0]
    dst = edge_index[1]

    def padn(a):
        return jnp.pad(a, ((0, N_PAD - N), (0, 0)))

    def pade(a, tot):
        return jnp.pad(a, (0, tot - a.shape[0]))

    xp = padn(x)
    xrp = padn(real)
    xip = padn(imag)
    src_p = pade(src, E_PAD + PF_PAD)
    dst_p = pade(dst, E_PAD + PF_PAD)
    ew_p = pade(edge_weight, E_PAD + PF_PAD)

    cos_e, sin_e, wsym_e = _tc(_prep_body, (_f32(1250, 128),) * 3)(
        edge_weight.reshape(1250, 128))
    cos_p = pade(cos_e.reshape(E), E_PAD + PF_PAD)
    sin_p = pade(sin_e.reshape(E), E_PAD + PF_PAD)
    wsym_p = pade(wsym_e.reshape(E), E_PAD + PF_PAD)

    degp = _sc_degrees(src_p, dst_p, wsym_p, ew_p)
    dm, dg, selfn = _tc(_dinv_body, (_f32(80, 128),) * 3)(
        degp.reshape(NC, 2, 80, 128))
    dm = dm.reshape(N_PAD // 128, 128)
    dg = dg.reshape(N_PAD // 128, 128)
    selfn = selfn.reshape(N_PAD, 1)

    nr_e, ni_e, nin_e, gn_e = _sc_norms(
        src_p, dst_p, wsym_p, ew_p, cos_p, sin_p, dm, dg)

    hrows = pade(jnp.concatenate([src, dst]), EH_PAD + PF_PAD)
    hcols = pade(jnp.concatenate([dst, src]), EH_PAD + PF_PAD)
    hnr = pade(jnp.concatenate([nr_e[:E], nr_e[:E]]), EH_PAD + PF_PAD)
    hni = pade(jnp.concatenate([ni_e[:E], nin_e[:E]]), EH_PAD + PF_PAD)

    w1cat = jnp.concatenate(
        [p['cheb1_W'][1], p['cheb1_W'][2], p['cheb1_W'][0]], axis=1)
    w2cat = jnp.concatenate(
        [p['cheb2_W'][1], p['cheb2_W'][2], p['cheb2_W'][0]], axis=1)
    w3cat = jnp.concatenate(
        [p['cheb3_W'][1], p['cheb3_W'][2], p['cheb3_W'][0]], axis=1)

    s1, a1 = _tc(_proj1_body, (_f32(N_PAD, 128), _f32(N_PAD, 64)))(
        xrp, xip, w1cat, p['cheb1_b'])
    p1 = _cprop[128](s1, hrows, hcols, hnr, hni)
    g1 = _tc(_mid_body_factory(32), _f32(N_PAD, 64))(p1)
    q1 = _cprop[64](g1, hrows, hcols, hnr, hni)

    s2, a2 = _tc(_proj_next_body_factory(32, 16),
                 (_f32(N_PAD, 64), _f32(N_PAD, 32)))(
        a1, p1, q1, w2cat, p['cheb2_b'])
    p2 = _cprop[64](s2, hrows, hcols, hnr, hni)
    g2 = _tc(_mid_body_factory(16), _f32(N_PAD, 32))(p2)
    q2 = _cprop[32](g2, hrows, hcols, hnr, hni)

    s3, a3 = _tc(_proj_next_body_factory(16, 8),
                 (_f32(N_PAD, 32), _f32(N_PAD, 16)))(
        a2, p2, q2, w3cat, p['cheb3_b'])
    p3 = _cprop[32](s3, hrows, hcols, hnr, hni)
    g3 = _tc(_mid_body_factory(8), _f32(N_PAD, 16))(p3)
    q3 = _cprop[16](g3, hrows, hcols, hnr, hni)

    x1p = _tc(_fin_cheb_body, _f32(N_PAD, 16))(a3, p3, q3)

    gn_dep = pade(gn_e, E_PAD + PF_PAD) + q3[0, 0, 0] * 0.0
    wg1 = jnp.concatenate([p['sgcn_W1'], p['cgcn_W1']], axis=1)
    sup1 = _tc(_gproj1_body, _f32(N_PAD, 128))(xp, wg1)
    gp1 = _rprop[128](sup1, dst_p, src_p, gn_dep, gn_dep)
    sup2 = _tc(_gnext_body_factory(64), _f32(N_PAD, 64))(
        gp1, sup1, selfn, p['sgcn_b1'], p['cgcn_b1'],
        p['sgcn_W2'], p['cgcn_W2'])
    gp2 = _rprop[64](sup2, dst_p, src_p, gn_e, gn_e)
    sup3 = _tc(_gnext_body_factory(32), _f32(N_PAD, 32))(
        gp2, sup2, selfn, p['sgcn_b2'], p['cgcn_b2'],
        p['sgcn_W3'], p['cgcn_W3'])
    gp3 = _rprop[32](sup3, dst_p, src_p, gn_e, gn_e)
    x2p, comp = _tc(_gfin_body, (_f32(N_PAD, 16), _f32(N_PAD, 16)))(
        gp3, sup3, selfn, p['sgcn_b3'], p['cgcn_b3'])

    logp_p, beta_p, emb_p = _tc(
        _attn_body, (_f32(N_PAD, 8), _f32(N_PAD, 3), _f32(N_PAD, 16)))(
        x1p, x2p, comp, p['attn_W1'], p['attn_b1'], p['attn_W2'],
        p['mlp_W1'], p['mlp_b1'], p['mlp_W2'], p['mlp_b2'],
        p['mlp_W3'], p['mlp_b3'])

    logp = logp_p[:N]
    beta = beta_p[:N, :, None]
    x1 = x1p[:N]
    com1 = comp[:N]
    x2 = x2p[:N]
    emb = emb_p[:N]
    return (logp, beta, x1, com1, com1, x2, emb)
